# baseline (speedup 1.0000x reference)
import jax
import jax.numpy as jnp
from jax.experimental import pallas as pl
from jax.experimental.pallas import tpu as pltpu

N = 100000
D = 128
DIM_OUT = 40
CH = 5000
NCH = N // CH
NBUF = 4


def _pipe_body(x_hbm, t_ref, o_hbm, lbuf, xsem, osem):
    for c in range(NCH):
        pltpu.make_async_copy(
            x_hbm.at[pl.ds(c * CH, CH), :],
            o_hbm.at[pl.ds(c * CH, CH), pl.ds(0, D)],
            xsem.at[c],
        ).start()

    cols = jax.lax.broadcasted_iota(jnp.int32, (CH, DIM_OUT), 1)
    for c in range(NCH):
        b = c % NBUF
        if c >= NBUF:
            pltpu.make_async_copy(
                lbuf.at[b],
                o_hbm.at[pl.ds((c - NBUF) * CH, CH), pl.ds(D, DIM_OUT)],
                osem.at[b],
            ).wait()
        tcol = t_ref[c, 0, :].reshape(CH, 1)
        lbuf[b] = (cols == tcol).astype(jnp.float32)
        pltpu.make_async_copy(
            lbuf.at[b],
            o_hbm.at[pl.ds(c * CH, CH), pl.ds(D, DIM_OUT)],
            osem.at[b],
        ).start()

    for c in range(NCH - NBUF, NCH):
        pltpu.make_async_copy(
            lbuf.at[c % NBUF],
            o_hbm.at[pl.ds(c * CH, CH), pl.ds(D, DIM_OUT)],
            osem.at[c % NBUF],
        ).wait()
    for c in range(NCH):
        pltpu.make_async_copy(
            x_hbm.at[pl.ds(c * CH, CH), :],
            o_hbm.at[pl.ds(c * CH, CH), pl.ds(0, D)],
            xsem.at[c],
        ).wait()


def kernel(x, y, train_mask):
    n = x.shape[0]
    t = jnp.where(train_mask, y[:, 0], -1).reshape(NCH, 1, CH)
    return pl.pallas_call(
        _pipe_body,
        in_specs=[
            pl.BlockSpec(memory_space=pltpu.MemorySpace.HBM),
            pl.BlockSpec(memory_space=pltpu.MemorySpace.VMEM),
        ],
        out_specs=pl.BlockSpec(memory_space=pltpu.MemorySpace.HBM),
        out_shape=jax.ShapeDtypeStruct((n, D + DIM_OUT), x.dtype),
        scratch_shapes=[
            pltpu.MemorySpace.VMEM((NBUF, CH, DIM_OUT), jnp.float32),
            pltpu.SemaphoreType.DMA((NCH,)),
            pltpu.SemaphoreType.DMA((NBUF,)),
        ],
    )(x, t)


# speedup vs baseline: 10.6930x; 10.6930x over previous
import functools

import jax
import jax.numpy as jnp
from jax import lax
from jax.experimental import pallas as pl
from jax.experimental.pallas import tpu as pltpu
from jax.experimental.pallas import tpu_sc as plsc

N = 100000
D = 128
DIM_OUT = 40
BLOCK = 5000
NB = N // BLOCK

_NC, _NS = 2, 16


def _sc_prep_body(y_ref, m_ref, t_ref, yv, mv, tv):
    u = lax.axis_index("s") * _NC + lax.axis_index("c")

    @pl.when(u < NB)
    def _():
        base = u * BLOCK
        pltpu.sync_copy(y_ref.at[pl.ds(base, BLOCK)], yv)
        pltpu.sync_copy(m_ref.at[pl.ds(base, BLOCK)], mv)
        for i in range(BLOCK // 16):
            sl = pl.ds(i * 16, 16)
            tv[sl] = jnp.where(mv[sl] != 0, yv[sl], -1)
        pltpu.sync_copy(tv, t_ref.at[u, 0, :])


_sc_prep = functools.partial(
    pl.kernel,
    out_type=jax.ShapeDtypeStruct((NB, 1, BLOCK), jnp.int32),
    mesh=plsc.VectorSubcoreMesh(
        core_axis_name="c", subcore_axis_name="s",
        num_cores=_NC, num_subcores=_NS,
    ),
    scratch_types=[
        pltpu.MemorySpace.VMEM((BLOCK,), jnp.int32),
        pltpu.MemorySpace.VMEM((BLOCK,), jnp.int32),
        pltpu.MemorySpace.VMEM((BLOCK,), jnp.int32),
    ],
)(_sc_prep_body)


def _encode_block(x_ref, t_ref, o_ref):
    o_ref[:, :D] = x_ref[...]
    tcol = t_ref[0, 0, :].reshape(BLOCK, 1)
    cols = jax.lax.broadcasted_iota(jnp.int32, (BLOCK, DIM_OUT), 1)
    o_ref[:, D:] = (cols == tcol).astype(jnp.float32)


def kernel(x, y, train_mask):
    n = x.shape[0]
    t = _sc_prep(y.reshape(n), train_mask.astype(jnp.int32))
    return pl.pallas_call(
        _encode_block,
        grid=(NB,),
        in_specs=[
            pl.BlockSpec((BLOCK, D), lambda i: (i, 0)),
            pl.BlockSpec((1, 1, BLOCK), lambda i: (i, 0, 0)),
        ],
        out_specs=pl.BlockSpec((BLOCK, D + DIM_OUT), lambda i: (i, 0)),
        out_shape=jax.ShapeDtypeStruct((n, D + DIM_OUT), x.dtype),
        compiler_params=pltpu.CompilerParams(
            dimension_semantics=("arbitrary",),
        ),
    )(x, t)


# speedup vs baseline: 10.6970x; 1.0004x over previous
import functools

import jax
import jax.numpy as jnp
from jax import lax
from jax.experimental import pallas as pl
from jax.experimental.pallas import tpu as pltpu
from jax.experimental.pallas import tpu_sc as plsc

N = 100000
D = 128
DIM_OUT = 40
BLOCK = 5000
NB = N // BLOCK

_NC, _NS = 2, 16


def _sc_prep_body(y_ref, m_ref, t_ref, yv, mv, tv):
    u = lax.axis_index("s") * _NC + lax.axis_index("c")

    @pl.when(u < NB)
    def _():
        base = u * BLOCK
        pltpu.sync_copy(y_ref.at[pl.ds(base, BLOCK)], yv)
        pltpu.sync_copy(m_ref.at[pl.ds(base, BLOCK)], mv)
        for i in range(-(-BLOCK // 16)):
            sl = pl.ds(min(i * 16, BLOCK - 16), 16)
            tv[sl] = jnp.where(mv[sl] != 0, yv[sl], -1)
        pltpu.sync_copy(tv, t_ref.at[u, 0, :])


_sc_prep = functools.partial(
    pl.kernel,
    out_type=jax.ShapeDtypeStruct((NB, 1, BLOCK), jnp.int32),
    mesh=plsc.VectorSubcoreMesh(
        core_axis_name="c", subcore_axis_name="s",
        num_cores=_NC, num_subcores=_NS,
    ),
    scratch_types=[
        pltpu.MemorySpace.VMEM((BLOCK,), jnp.int32),
        pltpu.MemorySpace.VMEM((BLOCK,), jnp.int32),
        pltpu.MemorySpace.VMEM((BLOCK,), jnp.int32),
    ],
)(_sc_prep_body)


def _encode_block(x_ref, t_ref, o_ref):
    o_ref[:, :D] = x_ref[...]
    tcol = t_ref[0, 0, :].reshape(BLOCK, 1)
    cols = jax.lax.broadcasted_iota(jnp.int32, (BLOCK, DIM_OUT), 1)
    o_ref[:, D:] = (cols == tcol).astype(jnp.float32)


def kernel(x, y, train_mask):
    n = x.shape[0]
    t = _sc_prep(y.reshape(n), train_mask.astype(jnp.int32))
    return pl.pallas_call(
        _encode_block,
        grid=(NB,),
        in_specs=[
            pl.BlockSpec((BLOCK, D), lambda i: (i, 0)),
            pl.BlockSpec((1, 1, BLOCK), lambda i: (i, 0, 0)),
        ],
        out_specs=pl.BlockSpec((BLOCK, D + DIM_OUT), lambda i: (i, 0)),
        out_shape=jax.ShapeDtypeStruct((n, D + DIM_OUT), x.dtype),
        compiler_params=pltpu.CompilerParams(
            dimension_semantics=("arbitrary",),
        ),
    )(x, t)
